# Initial kernel scaffold; baseline (speedup 1.0000x reference)
import functools

import jax
import jax.numpy as jnp
from jax import lax
from jax.experimental import pallas as pl
from jax.experimental.pallas import tpu as pltpu
from jax.experimental.pallas import tpu_sc as plsc

N = 10000
D = 128
E = 320000
EPS = 1e-5

NC, NS, L = 2, 16, 16
NW = NC * NS
NR = 10240
EPW = 10240
CH = 128
NCH = EPW // CH
STRIPE = NR // NS

_MESH = plsc.VectorSubcoreMesh(
    core_axis_name="c", subcore_axis_name="s", num_cores=NC, num_subcores=NS)


def _make_sc_scatter(with_counts):
  out_type = [jax.ShapeDtypeStruct((NC, NR, D), jnp.float32)]
  scratch = [
      pltpu.VMEM((NCH, CH), jnp.int32),
      pltpu.VMEM((NCH, CH), jnp.int32),
      pltpu.VMEM((CH, D), jnp.float32),
      pltpu.VMEM((CH, D), jnp.float32),
      pltpu.VMEM_SHARED((NR, D), jnp.float32),
      pltpu.SemaphoreType.DMA,
      pltpu.SemaphoreType.DMA,
  ]
  if with_counts:
    out_type.append(jax.ShapeDtypeStruct((NC, NR), jnp.float32))
    scratch += [
        pltpu.VMEM((NR,), jnp.float32),
        pltpu.VMEM_SHARED((NS, NR), jnp.float32),
        pltpu.VMEM((NS, STRIPE), jnp.float32),
        pltpu.VMEM((STRIPE,), jnp.float32),
    ]

  def body(x_hbm, src_hbm, dst_hbm, zrows_hbm, out_hbm, *rest):
    if with_counts:
      (cnt_hbm, src_v, dst_v, rows0, rows1, accum, sem0, sem1,
       cnt_buf, cnt_sh, credbuf, credout) = rest
    else:
      src_v, dst_v, rows0, rows1, accum, sem0, sem1 = rest
    c = lax.axis_index("c")
    s = lax.axis_index("s")
    wid = c * NS + s

    pltpu.sync_copy(src_hbm.at[wid], src_v)
    pltpu.sync_copy(dst_hbm.at[wid], dst_v)
    pltpu.sync_copy(zrows_hbm.at[pl.ds(s * STRIPE, STRIPE)],
                    accum.at[pl.ds(s * STRIPE, STRIPE)])
    pltpu.async_copy(x_hbm.at[src_v.at[0]], rows0, sem0)
    plsc.subcore_barrier()

    def step(i, carry):
      j0 = 2 * i
      j1 = j0 + 1
      pltpu.make_async_copy(x_hbm.at[src_v.at[j0]], rows0, sem0).wait()
      pltpu.async_copy(x_hbm.at[src_v.at[j1]], rows1, sem1)
      pltpu.sync_copy(rows0, accum.at[dst_v.at[j0]], add=True)
      pltpu.make_async_copy(x_hbm.at[src_v.at[j1]], rows1, sem1).wait()

      @pl.when(i + 1 < NCH // 2)
      def _():
        pltpu.async_copy(x_hbm.at[src_v.at[j0 + 2]], rows0, sem0)

      pltpu.sync_copy(rows1, accum.at[dst_v.at[j1]], add=True)
      return carry

    lax.fori_loop(0, NCH // 2, step, 0)

    if with_counts:
      zero16 = jnp.zeros((L,), jnp.float32)

      def zstep(i, carry):
        cnt_buf[pl.ds(i * L, L)] = zero16
        return carry

      lax.fori_loop(0, NR // L, zstep, 0)
      one16 = jnp.ones((L,), jnp.float32)

      def cstep(i, carry):
        j = i // (CH // L)
        k = i % (CH // L)
        dvec = dst_v[j, pl.ds(k * L, L)]
        plsc.addupdate_scatter(cnt_buf, [dvec], one16)
        return carry

      lax.fori_loop(0, EPW // L, cstep, 0)
      pltpu.sync_copy(cnt_buf, cnt_sh.at[s])

    plsc.subcore_barrier()

    pltpu.sync_copy(accum.at[pl.ds(s * STRIPE, STRIPE)],
                    out_hbm.at[c, pl.ds(s * STRIPE, STRIPE)])

    if with_counts:
      pltpu.sync_copy(cnt_sh.at[:, pl.ds(s * STRIPE, STRIPE)], credbuf)

      def rstep(k, carry):
        a = credbuf[0, pl.ds(k * L, L)]
        for r in range(1, NS):
          a = a + credbuf[r, pl.ds(k * L, L)]
        credout[pl.ds(k * L, L)] = a
        return carry

      lax.fori_loop(0, STRIPE // L, rstep, 0)
      pltpu.sync_copy(credout, cnt_hbm.at[c, pl.ds(s * STRIPE, STRIPE)])

  return pl.kernel(body, out_type=out_type, mesh=_MESH, scratch_types=scratch)


_sc_scatter_counts = _make_sc_scatter(True)
_sc_scatter = _make_sc_scatter(False)

BR = 1024
GRID = NR // BR


def _dense1_body(part, cnt, x, wl, wr, b, out):
  seg = part[0] + part[1]
  cv = cnt[...]
  ctot = jnp.maximum(cv[0] + cv[1], 1.0)
  mean = seg / ctot[:, None]
  h = jnp.dot(mean, wl[...], preferred_element_type=jnp.float32)
  h = h + jnp.dot(x[...], wr[...], preferred_element_type=jnp.float32)
  h = h + b[...]
  out[...] = jnp.maximum(h, 0.0)


def _dense2_body(part, cnt, h1, res, wl, wr, b, out):
  seg = part[0] + part[1]
  cv = cnt[...]
  ctot = jnp.maximum(cv[0] + cv[1], 1.0)
  mean = seg / ctot[:, None]
  h = jnp.dot(mean, wl[...], preferred_element_type=jnp.float32)
  h = h + jnp.dot(h1[...], wr[...], preferred_element_type=jnp.float32)
  h = h + b[...]
  out[...] = jnp.maximum(h, 0.0) + res[...]


_part_spec = pl.BlockSpec((NC, BR, D), lambda i: (0, i, 0))
_cnt_spec = pl.BlockSpec((NC, BR), lambda i: (0, i))
_row_spec = pl.BlockSpec((BR, D), lambda i: (i, 0))
_w_spec = pl.BlockSpec((D, D), lambda i: (0, 0))
_b_spec = pl.BlockSpec((1, D), lambda i: (0, 0))

_dense1 = pl.pallas_call(
    _dense1_body,
    grid=(GRID,),
    in_specs=[_part_spec, _cnt_spec, _row_spec, _w_spec, _w_spec, _b_spec],
    out_specs=_row_spec,
    out_shape=jax.ShapeDtypeStruct((N, D), jnp.float32),
)

_dense2 = pl.pallas_call(
    _dense2_body,
    grid=(GRID,),
    in_specs=[_part_spec, _cnt_spec, _row_spec, _row_spec, _w_spec, _w_spec,
              _b_spec],
    out_specs=_row_spec,
    out_shape=jax.ShapeDtypeStruct((N, D), jnp.float32),
)


def kernel(x, edge_index, W1l, b1, W1r, W2l, b2, W2r, g1, be1, g2, be2):
  s1 = g1 / jnp.sqrt(1.0 + EPS)
  s2 = g2 / jnp.sqrt(1.0 + EPS)
  w1l = W1l * s1[None, :]
  w1r = W1r * s1[None, :]
  bb1 = (b1 * s1 + be1)[None, :]
  w2l = W2l * s2[None, :]
  w2r = W2r * s2[None, :]
  bb2 = (b2 * s2 + be2)[None, :]

  src = edge_index[0]
  dst = edge_index[1]
  pad = NW * EPW - E
  src_p = jnp.concatenate(
      [src, jnp.zeros((pad,), jnp.int32)]).reshape(NW, NCH, CH)
  dst_p = jnp.concatenate(
      [dst, N + (jnp.arange(pad, dtype=jnp.int32) % L)]).reshape(NW, NCH, CH)
  zrows = jnp.zeros((NR, D), jnp.float32)

  part1, cnt = _sc_scatter_counts(x, src_p, dst_p, zrows)
  h1 = _dense1(part1, cnt, x, w1l, w1r, bb1)
  (part2,) = _sc_scatter(h1, src_p, dst_p, zrows)
  out = _dense2(part2, cnt, h1, x, w2l, w2r, bb2)
  return out


# speedup vs baseline: 3.4344x; 3.4344x over previous
import functools

import jax
import jax.numpy as jnp
from jax import lax
from jax.experimental import pallas as pl
from jax.experimental.pallas import tpu as pltpu
from jax.experimental.pallas import tpu_sc as plsc

N = 10000
D = 128
E = 320000
EPS = 1e-5

NC, NS, L = 2, 16, 16
NW = NC * NS
NR = 10240
EPW = 10240
CH = 128
NCH = EPW // CH
NH = 2
HCH = NCH // NH
STRIPE = NR // NS
CW = 16

_MESH = plsc.VectorSubcoreMesh(
    core_axis_name="c", subcore_axis_name="s", num_cores=NC, num_subcores=NS)


def _sc_scatter_body(x_hbm, src_hbm, dst_hbm, zrows_hbm, out_hbm,
                     src_v, dst_v, rows0, rows1, accum, sem0, sem1):
  c = lax.axis_index("c")
  s = lax.axis_index("s")
  wid = c * NS + s

  pltpu.sync_copy(zrows_hbm.at[pl.ds(s * STRIPE, STRIPE)],
                  accum.at[pl.ds(s * STRIPE, STRIPE)])

  def step(i, carry):
    j0 = 2 * i
    j1 = j0 + 1
    pltpu.make_async_copy(x_hbm.at[src_v.at[j0]], rows0, sem0).wait()
    pltpu.async_copy(x_hbm.at[src_v.at[j1]], rows1, sem1)
    pltpu.sync_copy(rows0, accum.at[dst_v.at[j0]], add=True)
    pltpu.make_async_copy(x_hbm.at[src_v.at[j1]], rows1, sem1).wait()

    @pl.when(i + 1 < HCH // 2)
    def _():
      pltpu.async_copy(x_hbm.at[src_v.at[j0 + 2]], rows0, sem0)

    pltpu.sync_copy(rows1, accum.at[dst_v.at[j1]], add=True)
    return carry

  for h in range(NH):
    pltpu.sync_copy(src_hbm.at[wid, pl.ds(h * HCH, HCH)], src_v)
    pltpu.sync_copy(dst_hbm.at[wid, pl.ds(h * HCH, HCH)], dst_v)
    pltpu.async_copy(x_hbm.at[src_v.at[0]], rows0, sem0)
    if h == 0:
      plsc.subcore_barrier()
    lax.fori_loop(0, HCH // 2, step, 0)

  plsc.subcore_barrier()

  pltpu.sync_copy(accum.at[pl.ds(s * STRIPE, STRIPE)],
                  out_hbm.at[c, pl.ds(s * STRIPE, STRIPE)])


_sc_scatter = pl.kernel(
    _sc_scatter_body,
    out_type=[jax.ShapeDtypeStruct((NC, NR, D), jnp.float32)],
    mesh=_MESH,
    scratch_types=[
        pltpu.VMEM((HCH, CH), jnp.int32),
        pltpu.VMEM((HCH, CH), jnp.int32),
        pltpu.VMEM((CH, D), jnp.float32),
        pltpu.VMEM((CH, D), jnp.float32),
        pltpu.VMEM_SHARED((NR, D), jnp.float32),
        pltpu.SemaphoreType.DMA,
        pltpu.SemaphoreType.DMA,
    ])


def _sc_counts_body(dst_hbm, cnt_hbm, dst_v, cnt_buf, cnt_sh, credbuf, credout):
  c = lax.axis_index("c")
  s = lax.axis_index("s")
  wid = c * NS + s
  pltpu.sync_copy(dst_hbm.at[wid], dst_v)
  zero16 = jnp.zeros((L,), jnp.float32)

  def zstep(i, carry):
    cnt_buf[pl.ds(i * L, L)] = zero16
    return carry

  lax.fori_loop(0, NR // L, zstep, 0)
  one16 = jnp.ones((L,), jnp.float32)

  def cstep(i, carry):
    j = i // (CH // L)
    k = i % (CH // L)
    dvec = dst_v[j, pl.ds(k * L, L)]
    plsc.addupdate_scatter(cnt_buf, [dvec], one16)
    return carry

  lax.fori_loop(0, EPW // L, cstep, 0)
  pltpu.sync_copy(cnt_buf, cnt_sh.at[s])
  plsc.subcore_barrier()

  pltpu.sync_copy(cnt_sh.at[:, pl.ds(s * STRIPE, STRIPE)], credbuf)

  def rstep(k, carry):
    a = credbuf[0, pl.ds(k * L, L)]
    for r in range(1, NS):
      a = a + credbuf[r, pl.ds(k * L, L)]
    credout[pl.ds(k * L, L)] = a
    return carry

  lax.fori_loop(0, STRIPE // L, rstep, 0)
  pltpu.sync_copy(credout, cnt_hbm.at[c, pl.ds(s * STRIPE, STRIPE)])


_sc_counts = pl.kernel(
    _sc_counts_body,
    out_type=[jax.ShapeDtypeStruct((NC, NR), jnp.float32)],
    mesh=_MESH,
    scratch_types=[
        pltpu.VMEM((NCH, CH), jnp.int32),
        pltpu.VMEM((NR,), jnp.float32),
        pltpu.VMEM_SHARED((NS, NR), jnp.float32),
        pltpu.VMEM((NS, STRIPE), jnp.float32),
        pltpu.VMEM((STRIPE,), jnp.float32),
    ],
    compiler_params=pltpu.CompilerParams(needs_layout_passes=False))

BR = 1024
GRID = NR // BR


def _dense1_body(part, cnt, x, wl, wr, b, out):
  seg = part[0] + part[1]
  cv = cnt[...]
  ctot = jnp.maximum(cv[0] + cv[1], 1.0)
  mean = seg / ctot[:, None]
  h = jnp.dot(mean, wl[...], preferred_element_type=jnp.float32)
  h = h + jnp.dot(x[...], wr[...], preferred_element_type=jnp.float32)
  h = h + b[...]
  out[...] = jnp.maximum(h, 0.0)


def _dense2_body(part, cnt, h1, res, wl, wr, b, out):
  seg = part[0] + part[1]
  cv = cnt[...]
  ctot = jnp.maximum(cv[0] + cv[1], 1.0)
  mean = seg / ctot[:, None]
  h = jnp.dot(mean, wl[...], preferred_element_type=jnp.float32)
  h = h + jnp.dot(h1[...], wr[...], preferred_element_type=jnp.float32)
  h = h + b[...]
  out[...] = jnp.maximum(h, 0.0) + res[...]


_part_spec = pl.BlockSpec((NC, BR, D), lambda i: (0, i, 0))
_cnt_spec = pl.BlockSpec((NC, BR), lambda i: (0, i))
_row_spec = pl.BlockSpec((BR, D), lambda i: (i, 0))
_w_spec = pl.BlockSpec((D, D), lambda i: (0, 0))
_b_spec = pl.BlockSpec((1, D), lambda i: (0, 0))

_dense1 = pl.pallas_call(
    _dense1_body,
    grid=(GRID,),
    in_specs=[_part_spec, _cnt_spec, _row_spec, _w_spec, _w_spec, _b_spec],
    out_specs=_row_spec,
    out_shape=jax.ShapeDtypeStruct((N, D), jnp.float32),
)

_dense2 = pl.pallas_call(
    _dense2_body,
    grid=(GRID,),
    in_specs=[_part_spec, _cnt_spec, _row_spec, _row_spec, _w_spec, _w_spec,
              _b_spec],
    out_specs=_row_spec,
    out_shape=jax.ShapeDtypeStruct((N, D), jnp.float32),
)


def kernel(x, edge_index, W1l, b1, W1r, W2l, b2, W2r, g1, be1, g2, be2):
  s1 = g1 / jnp.sqrt(1.0 + EPS)
  s2 = g2 / jnp.sqrt(1.0 + EPS)
  w1l = W1l * s1[None, :]
  w1r = W1r * s1[None, :]
  bb1 = (b1 * s1 + be1)[None, :]
  w2l = W2l * s2[None, :]
  w2r = W2r * s2[None, :]
  bb2 = (b2 * s2 + be2)[None, :]

  src = edge_index[0]
  dst = edge_index[1]
  pad = NW * EPW - E
  src_p = jnp.concatenate(
      [src, jnp.zeros((pad,), jnp.int32)]).reshape(NW, NCH, CH)
  dst_p = jnp.concatenate(
      [dst, N + (jnp.arange(pad, dtype=jnp.int32) % L)]).reshape(NW, NCH, CH)
  zrows = jnp.zeros((NR, D), jnp.float32)

  (cnt,) = _sc_counts(dst_p)
  (part1,) = _sc_scatter(x, src_p, dst_p, zrows)
  h1 = _dense1(part1, cnt, x, w1l, w1r, bb1)
  (part2,) = _sc_scatter(h1, src_p, dst_p, zrows)
  out = _dense2(part2, cnt, h1, x, w2l, w2r, bb2)
  return out


# speedup vs baseline: 3.6801x; 1.0715x over previous
import functools

import jax
import jax.numpy as jnp
from jax import lax
from jax.experimental import pallas as pl
from jax.experimental.pallas import tpu as pltpu
from jax.experimental.pallas import tpu_sc as plsc

N = 10000
D = 128
E = 320000
EPS = 1e-5

NC, NS, L = 2, 16, 16
NW = NC * NS
NR = 10240
EPW = 10240
CH = 80
NCH = EPW // CH
NSTAGE = 4
SCH = NCH // NSTAGE
NBUF = 4
NGRP = SCH // NBUF
STRIPE = NR // NS

_MESH = plsc.VectorSubcoreMesh(
    core_axis_name="c", subcore_axis_name="s", num_cores=NC, num_subcores=NS)


def _sc_scatter_body(x_hbm, src_hbm, dst_hbm, zrows_hbm, out_hbm,
                     src_v, dst_v, buf0, buf1, buf2, buf3, accum,
                     g0, g1, g2, g3, s0, s1, s2, s3):
  bufs = (buf0, buf1, buf2, buf3)
  gsem = (g0, g1, g2, g3)
  ssem = (s0, s1, s2, s3)
  c = lax.axis_index("c")
  s = lax.axis_index("s")
  wid = c * NS + s

  pltpu.sync_copy(zrows_hbm.at[pl.ds(s * STRIPE, STRIPE)],
                  accum.at[pl.ds(s * STRIPE, STRIPE)])

  def grp(g, carry):
    for b in range(NBUF):
      j = g * NBUF + b
      pltpu.make_async_copy(x_hbm.at[src_v.at[j]], bufs[b], gsem[b]).wait()
      pltpu.async_copy(bufs[b], accum.at[dst_v.at[j]], ssem[b], add=True)
    for b in range(NBUF):
      j = g * NBUF + b

      def _advance(b=b, j=j):
        pltpu.make_async_copy(bufs[b], accum.at[dst_v.at[j]], ssem[b]).wait()
        pltpu.async_copy(x_hbm.at[src_v.at[j + NBUF]], bufs[b], gsem[b])

      pl.when(g + 1 < NGRP)(_advance)
    return carry

  for st in range(NSTAGE):
    pltpu.sync_copy(src_hbm.at[wid, pl.ds(st * SCH, SCH)], src_v)
    pltpu.sync_copy(dst_hbm.at[wid, pl.ds(st * SCH, SCH)], dst_v)
    for b in range(NBUF):
      pltpu.async_copy(x_hbm.at[src_v.at[b]], bufs[b], gsem[b])
    if st == 0:
      plsc.subcore_barrier()
    lax.fori_loop(0, NGRP, grp, 0)
    for b in range(NBUF):
      j = SCH - NBUF + b
      pltpu.make_async_copy(bufs[b], accum.at[dst_v.at[j]], ssem[b]).wait()

  plsc.subcore_barrier()

  pltpu.sync_copy(accum.at[pl.ds(s * STRIPE, STRIPE)],
                  out_hbm.at[c, pl.ds(s * STRIPE, STRIPE)])


_sc_scatter = pl.kernel(
    _sc_scatter_body,
    out_type=[jax.ShapeDtypeStruct((NC, NR, D), jnp.float32)],
    mesh=_MESH,
    scratch_types=[
        pltpu.VMEM((SCH, CH), jnp.int32),
        pltpu.VMEM((SCH, CH), jnp.int32),
        pltpu.VMEM((CH, D), jnp.float32),
        pltpu.VMEM((CH, D), jnp.float32),
        pltpu.VMEM((CH, D), jnp.float32),
        pltpu.VMEM((CH, D), jnp.float32),
        pltpu.VMEM_SHARED((NR, D), jnp.float32),
        pltpu.SemaphoreType.DMA,
        pltpu.SemaphoreType.DMA,
        pltpu.SemaphoreType.DMA,
        pltpu.SemaphoreType.DMA,
        pltpu.SemaphoreType.DMA,
        pltpu.SemaphoreType.DMA,
        pltpu.SemaphoreType.DMA,
        pltpu.SemaphoreType.DMA,
    ])


def _sc_counts_body(dst_hbm, cnt_hbm, dst_v, cnt_buf, cnt_sh, credbuf, credout):
  c = lax.axis_index("c")
  s = lax.axis_index("s")
  wid = c * NS + s
  pltpu.sync_copy(dst_hbm.at[wid], dst_v)
  zero16 = jnp.zeros((L,), jnp.float32)

  def zstep(i, carry):
    cnt_buf[pl.ds(i * L, L)] = zero16
    return carry

  lax.fori_loop(0, NR // L, zstep, 0)
  one16 = jnp.ones((L,), jnp.float32)

  def cstep(i, carry):
    j = i // (CH // L)
    k = i % (CH // L)
    dvec = dst_v[j, pl.ds(k * L, L)]
    plsc.addupdate_scatter(cnt_buf, [dvec], one16)
    return carry

  lax.fori_loop(0, EPW // L, cstep, 0)
  pltpu.sync_copy(cnt_buf, cnt_sh.at[s])
  plsc.subcore_barrier()

  pltpu.sync_copy(cnt_sh.at[:, pl.ds(s * STRIPE, STRIPE)], credbuf)

  def rstep(k, carry):
    a = credbuf[0, pl.ds(k * L, L)]
    for r in range(1, NS):
      a = a + credbuf[r, pl.ds(k * L, L)]
    credout[pl.ds(k * L, L)] = a
    return carry

  lax.fori_loop(0, STRIPE // L, rstep, 0)
  pltpu.sync_copy(credout, cnt_hbm.at[c, pl.ds(s * STRIPE, STRIPE)])


_sc_counts = pl.kernel(
    _sc_counts_body,
    out_type=[jax.ShapeDtypeStruct((NC, NR), jnp.float32)],
    mesh=_MESH,
    scratch_types=[
        pltpu.VMEM((NCH, CH), jnp.int32),
        pltpu.VMEM((NR,), jnp.float32),
        pltpu.VMEM_SHARED((NS, NR), jnp.float32),
        pltpu.VMEM((NS, STRIPE), jnp.float32),
        pltpu.VMEM((STRIPE,), jnp.float32),
    ],
    compiler_params=pltpu.CompilerParams(needs_layout_passes=False))

BR = 1024
GRID = NR // BR


def _dense1_body(part, cnt, x, wl, wr, b, out):
  seg = part[0] + part[1]
  cv = cnt[...]
  ctot = jnp.maximum(cv[0] + cv[1], 1.0)
  mean = seg / ctot[:, None]
  h = jnp.dot(mean, wl[...], preferred_element_type=jnp.float32)
  h = h + jnp.dot(x[...], wr[...], preferred_element_type=jnp.float32)
  h = h + b[...]
  out[...] = jnp.maximum(h, 0.0)


def _dense2_body(part, cnt, h1, res, wl, wr, b, out):
  seg = part[0] + part[1]
  cv = cnt[...]
  ctot = jnp.maximum(cv[0] + cv[1], 1.0)
  mean = seg / ctot[:, None]
  h = jnp.dot(mean, wl[...], preferred_element_type=jnp.float32)
  h = h + jnp.dot(h1[...], wr[...], preferred_element_type=jnp.float32)
  h = h + b[...]
  out[...] = jnp.maximum(h, 0.0) + res[...]


_part_spec = pl.BlockSpec((NC, BR, D), lambda i: (0, i, 0))
_cnt_spec = pl.BlockSpec((NC, BR), lambda i: (0, i))
_row_spec = pl.BlockSpec((BR, D), lambda i: (i, 0))
_w_spec = pl.BlockSpec((D, D), lambda i: (0, 0))
_b_spec = pl.BlockSpec((1, D), lambda i: (0, 0))

_dense1 = pl.pallas_call(
    _dense1_body,
    grid=(GRID,),
    in_specs=[_part_spec, _cnt_spec, _row_spec, _w_spec, _w_spec, _b_spec],
    out_specs=_row_spec,
    out_shape=jax.ShapeDtypeStruct((N, D), jnp.float32),
)

_dense2 = pl.pallas_call(
    _dense2_body,
    grid=(GRID,),
    in_specs=[_part_spec, _cnt_spec, _row_spec, _row_spec, _w_spec, _w_spec,
              _b_spec],
    out_specs=_row_spec,
    out_shape=jax.ShapeDtypeStruct((N, D), jnp.float32),
)


def kernel(x, edge_index, W1l, b1, W1r, W2l, b2, W2r, g1, be1, g2, be2):
  s1 = g1 / jnp.sqrt(1.0 + EPS)
  s2 = g2 / jnp.sqrt(1.0 + EPS)
  w1l = W1l * s1[None, :]
  w1r = W1r * s1[None, :]
  bb1 = (b1 * s1 + be1)[None, :]
  w2l = W2l * s2[None, :]
  w2r = W2r * s2[None, :]
  bb2 = (b2 * s2 + be2)[None, :]

  src = edge_index[0]
  dst = edge_index[1]
  pad = NW * EPW - E
  src_p = jnp.concatenate(
      [src, jnp.zeros((pad,), jnp.int32)]).reshape(NW, NCH, CH)
  dst_p = jnp.concatenate(
      [dst, N + (jnp.arange(pad, dtype=jnp.int32) % L)]).reshape(NW, NCH, CH)
  zrows = jnp.zeros((NR, D), jnp.float32)

  (cnt,) = _sc_counts(dst_p)
  (part1,) = _sc_scatter(x, src_p, dst_p, zrows)
  h1 = _dense1(part1, cnt, x, w1l, w1r, bb1)
  (part2,) = _sc_scatter(h1, src_p, dst_p, zrows)
  out = _dense2(part2, cnt, h1, x, w2l, w2r, bb2)
  return out


# speedup vs baseline: 4.0538x; 1.1015x over previous
import functools

import jax
import jax.numpy as jnp
from jax import lax
from jax.experimental import pallas as pl
from jax.experimental.pallas import tpu as pltpu
from jax.experimental.pallas import tpu_sc as plsc

N = 10000
D = 128
E = 320000
EPS = 1e-5

NC, NS, L = 2, 16, 16
NW = NC * NS
NR = 10240
EPW = 10240
CH = 64
TOTCH = NW * EPW // CH
SST = 32
NBUF = 4
K0 = 256
K1 = 64
NCHC = EPW // CH
STRIPE = NR // NS

_MESH = plsc.VectorSubcoreMesh(
    core_axis_name="c", subcore_axis_name="s", num_cores=NC, num_subcores=NS)


def _sc_scatter_body(x_hbm, src_hbm, dst_hbm, zrows_hbm, out_hbm,
                     src_v, dst_v, buf0, buf1, buf2, buf3, accum,
                     g0, g1, g2, g3, s0, s1, s2, s3):
  bufs = (buf0, buf1, buf2, buf3)
  gsem = (g0, g1, g2, g3)
  ssem = (s0, s1, s2, s3)
  c = lax.axis_index("c")
  s = lax.axis_index("s")

  pltpu.sync_copy(zrows_hbm.at[pl.ds(s * STRIPE, STRIPE)],
                  accum.at[pl.ds(s * STRIPE, STRIPE)])
  plsc.subcore_barrier()

  def grp(g, carry):
    for b in range(NBUF):
      j = g * NBUF + b
      pltpu.make_async_copy(x_hbm.at[src_v.at[j]], bufs[b], gsem[b]).wait()
      pltpu.async_copy(bufs[b], accum.at[dst_v.at[j]], ssem[b], add=True)
    for b in range(NBUF):
      j = g * NBUF + b

      def _advance(b=b, j=j):
        pltpu.make_async_copy(bufs[b], accum.at[dst_v.at[j]], ssem[b]).wait()
        pltpu.async_copy(x_hbm.at[src_v.at[j + NBUF]], bufs[b], gsem[b])

      pl.when(g + 1 < SST // NBUF)(_advance)
    return carry

  def run(base, nst):
    for st in range(nst):
      sb = base + st * SST
      pltpu.sync_copy(src_hbm.at[pl.ds(sb, SST)], src_v)
      pltpu.sync_copy(dst_hbm.at[pl.ds(sb, SST)], dst_v)
      for b in range(NBUF):
        pltpu.async_copy(x_hbm.at[src_v.at[b]], bufs[b], gsem[b])
      lax.fori_loop(0, SST // NBUF, grp, 0)
      for b in range(NBUF):
        j = SST - NBUF + b
        pltpu.make_async_copy(bufs[b], accum.at[dst_v.at[j]], ssem[b]).wait()

  pl.when(c == 0)(lambda: run(s * K0, K0 // SST))
  pl.when(c == 1)(lambda: run(NS * K0 + s * K1, K1 // SST))

  plsc.subcore_barrier()

  pltpu.sync_copy(accum.at[pl.ds(s * STRIPE, STRIPE)],
                  out_hbm.at[c, pl.ds(s * STRIPE, STRIPE)])


_sc_scatter = pl.kernel(
    _sc_scatter_body,
    out_type=[jax.ShapeDtypeStruct((NC, NR, D), jnp.float32)],
    mesh=_MESH,
    scratch_types=[
        pltpu.VMEM((SST, CH), jnp.int32),
        pltpu.VMEM((SST, CH), jnp.int32),
        pltpu.VMEM((CH, D), jnp.float32),
        pltpu.VMEM((CH, D), jnp.float32),
        pltpu.VMEM((CH, D), jnp.float32),
        pltpu.VMEM((CH, D), jnp.float32),
        pltpu.VMEM_SHARED((NR, D), jnp.float32),
        pltpu.SemaphoreType.DMA,
        pltpu.SemaphoreType.DMA,
        pltpu.SemaphoreType.DMA,
        pltpu.SemaphoreType.DMA,
        pltpu.SemaphoreType.DMA,
        pltpu.SemaphoreType.DMA,
        pltpu.SemaphoreType.DMA,
        pltpu.SemaphoreType.DMA,
    ])


def _sc_counts_body(dst_hbm, cnt_hbm, dst_v, cnt_buf, cnt_sh, credbuf, credout):
  c = lax.axis_index("c")
  s = lax.axis_index("s")
  wid = c * NS + s
  pltpu.sync_copy(dst_hbm.at[wid], dst_v)
  zero16 = jnp.zeros((L,), jnp.float32)

  def zstep(i, carry):
    cnt_buf[pl.ds(i * L, L)] = zero16
    return carry

  lax.fori_loop(0, NR // L, zstep, 0)
  one16 = jnp.ones((L,), jnp.float32)

  def cstep(i, carry):
    j = i // (CH // L)
    k = i % (CH // L)
    dvec = dst_v[j, pl.ds(k * L, L)]
    plsc.addupdate_scatter(cnt_buf, [dvec], one16)
    return carry

  lax.fori_loop(0, EPW // L, cstep, 0)
  pltpu.sync_copy(cnt_buf, cnt_sh.at[s])
  plsc.subcore_barrier()

  pltpu.sync_copy(cnt_sh.at[:, pl.ds(s * STRIPE, STRIPE)], credbuf)

  def rstep(k, carry):
    a = credbuf[0, pl.ds(k * L, L)]
    for r in range(1, NS):
      a = a + credbuf[r, pl.ds(k * L, L)]
    credout[pl.ds(k * L, L)] = a
    return carry

  lax.fori_loop(0, STRIPE // L, rstep, 0)
  pltpu.sync_copy(credout, cnt_hbm.at[c, pl.ds(s * STRIPE, STRIPE)])


_sc_counts = pl.kernel(
    _sc_counts_body,
    out_type=[jax.ShapeDtypeStruct((NC, NR), jnp.float32)],
    mesh=_MESH,
    scratch_types=[
        pltpu.VMEM((NCHC, CH), jnp.int32),
        pltpu.VMEM((NR,), jnp.float32),
        pltpu.VMEM_SHARED((NS, NR), jnp.float32),
        pltpu.VMEM((NS, STRIPE), jnp.float32),
        pltpu.VMEM((STRIPE,), jnp.float32),
    ],
    compiler_params=pltpu.CompilerParams(needs_layout_passes=False))

BR = 1024
GRID = NR // BR


def _dense1_body(part, cnt, x, wl, wr, b, out):
  seg = part[0] + part[1]
  cv = cnt[...]
  ctot = jnp.maximum(cv[0] + cv[1], 1.0)
  mean = seg / ctot[:, None]
  h = jnp.dot(mean, wl[...], preferred_element_type=jnp.float32)
  h = h + jnp.dot(x[...], wr[...], preferred_element_type=jnp.float32)
  h = h + b[...]
  out[...] = jnp.maximum(h, 0.0)


def _dense2_body(part, cnt, h1, res, wl, wr, b, out):
  seg = part[0] + part[1]
  cv = cnt[...]
  ctot = jnp.maximum(cv[0] + cv[1], 1.0)
  mean = seg / ctot[:, None]
  h = jnp.dot(mean, wl[...], preferred_element_type=jnp.float32)
  h = h + jnp.dot(h1[...], wr[...], preferred_element_type=jnp.float32)
  h = h + b[...]
  out[...] = jnp.maximum(h, 0.0) + res[...]


_part_spec = pl.BlockSpec((NC, BR, D), lambda i: (0, i, 0))
_cnt_spec = pl.BlockSpec((NC, BR), lambda i: (0, i))
_row_spec = pl.BlockSpec((BR, D), lambda i: (i, 0))
_w_spec = pl.BlockSpec((D, D), lambda i: (0, 0))
_b_spec = pl.BlockSpec((1, D), lambda i: (0, 0))

_dense1 = pl.pallas_call(
    _dense1_body,
    grid=(GRID,),
    in_specs=[_part_spec, _cnt_spec, _row_spec, _w_spec, _w_spec, _b_spec],
    out_specs=_row_spec,
    out_shape=jax.ShapeDtypeStruct((N, D), jnp.float32),
)

_dense2 = pl.pallas_call(
    _dense2_body,
    grid=(GRID,),
    in_specs=[_part_spec, _cnt_spec, _row_spec, _row_spec, _w_spec, _w_spec,
              _b_spec],
    out_specs=_row_spec,
    out_shape=jax.ShapeDtypeStruct((N, D), jnp.float32),
)


def kernel(x, edge_index, W1l, b1, W1r, W2l, b2, W2r, g1, be1, g2, be2):
  s1 = g1 / jnp.sqrt(1.0 + EPS)
  s2 = g2 / jnp.sqrt(1.0 + EPS)
  w1l = W1l * s1[None, :]
  w1r = W1r * s1[None, :]
  bb1 = (b1 * s1 + be1)[None, :]
  w2l = W2l * s2[None, :]
  w2r = W2r * s2[None, :]
  bb2 = (b2 * s2 + be2)[None, :]

  src = edge_index[0]
  dst = edge_index[1]
  pad = NW * EPW - E
  src_p = jnp.concatenate([src, jnp.zeros((pad,), jnp.int32)])
  dst_p = jnp.concatenate([dst, N + (jnp.arange(pad, dtype=jnp.int32) % L)])
  src_f = src_p.reshape(TOTCH, CH)
  dst_f = dst_p.reshape(TOTCH, CH)
  dst_c = dst_p.reshape(NW, NCHC, CH)
  zrows = jnp.zeros((NR, D), jnp.float32)

  (cnt,) = _sc_counts(dst_c)
  (part1,) = _sc_scatter(x, src_f, dst_f, zrows)
  h1 = _dense1(part1, cnt, x, w1l, w1r, bb1)
  (part2,) = _sc_scatter(h1, src_f, dst_f, zrows)
  out = _dense2(part2, cnt, h1, x, w2l, w2r, bb2)
  return out


# speedup vs baseline: 4.6234x; 1.1405x over previous
import functools

import jax
import jax.numpy as jnp
from jax import lax
from jax.experimental import pallas as pl
from jax.experimental.pallas import tpu as pltpu
from jax.experimental.pallas import tpu_sc as plsc

N = 10000
D = 128
E = 320000
EPS = 1e-5

NC, NS, L = 2, 16, 16
NW = NC * NS
NR = 10240
EPW = 10240
CH = 64
TOTCH = NW * EPW // CH
SST = 32
NBUF = 4
K0 = 288
K1 = 32
NCHC = EPW // CH
STRIPE = NR // NS

_MESH = plsc.VectorSubcoreMesh(
    core_axis_name="c", subcore_axis_name="s", num_cores=NC, num_subcores=NS)


def _sc_scatter_body(x_hbm, src_hbm, dst_hbm, zrows_hbm, out_hbm,
                     src_v, dst_v, buf0, buf1, buf2, buf3, accum,
                     g0, g1, g2, g3, s0, s1, s2, s3):
  bufs = (buf0, buf1, buf2, buf3)
  gsem = (g0, g1, g2, g3)
  ssem = (s0, s1, s2, s3)
  c = lax.axis_index("c")
  s = lax.axis_index("s")

  pltpu.sync_copy(zrows_hbm.at[pl.ds(s * STRIPE, STRIPE)],
                  accum.at[pl.ds(s * STRIPE, STRIPE)])
  plsc.subcore_barrier()

  def grp(g, carry):
    for b in range(NBUF):
      j = g * NBUF + b
      pltpu.make_async_copy(x_hbm.at[src_v.at[j]], bufs[b], gsem[b]).wait()
      pltpu.async_copy(bufs[b], accum.at[dst_v.at[j]], ssem[b], add=True)
    for b in range(NBUF):
      j = g * NBUF + b

      def _advance(b=b, j=j):
        pltpu.make_async_copy(bufs[b], accum.at[dst_v.at[j]], ssem[b]).wait()
        pltpu.async_copy(x_hbm.at[src_v.at[j + NBUF]], bufs[b], gsem[b])

      pl.when(g + 1 < SST // NBUF)(_advance)
    return carry

  def run(base, nst):
    for st in range(nst):
      sb = base + st * SST
      pltpu.sync_copy(src_hbm.at[pl.ds(sb, SST)], src_v)
      pltpu.sync_copy(dst_hbm.at[pl.ds(sb, SST)], dst_v)
      for b in range(NBUF):
        pltpu.async_copy(x_hbm.at[src_v.at[b]], bufs[b], gsem[b])
      lax.fori_loop(0, SST // NBUF, grp, 0)
      for b in range(NBUF):
        j = SST - NBUF + b
        pltpu.make_async_copy(bufs[b], accum.at[dst_v.at[j]], ssem[b]).wait()

  pl.when(c == 0)(lambda: run(s * K0, K0 // SST))
  pl.when(c == 1)(lambda: run(NS * K0 + s * K1, K1 // SST))

  plsc.subcore_barrier()

  pltpu.sync_copy(accum.at[pl.ds(s * STRIPE, STRIPE)],
                  out_hbm.at[c, pl.ds(s * STRIPE, STRIPE)])


_sc_scatter = pl.kernel(
    _sc_scatter_body,
    out_type=[jax.ShapeDtypeStruct((NC, NR, D), jnp.float32)],
    mesh=_MESH,
    scratch_types=[
        pltpu.VMEM((SST, CH), jnp.int32),
        pltpu.VMEM((SST, CH), jnp.int32),
        pltpu.VMEM((CH, D), jnp.float32),
        pltpu.VMEM((CH, D), jnp.float32),
        pltpu.VMEM((CH, D), jnp.float32),
        pltpu.VMEM((CH, D), jnp.float32),
        pltpu.VMEM_SHARED((NR, D), jnp.float32),
        pltpu.SemaphoreType.DMA,
        pltpu.SemaphoreType.DMA,
        pltpu.SemaphoreType.DMA,
        pltpu.SemaphoreType.DMA,
        pltpu.SemaphoreType.DMA,
        pltpu.SemaphoreType.DMA,
        pltpu.SemaphoreType.DMA,
        pltpu.SemaphoreType.DMA,
    ])


def _sc_counts_body(dst_hbm, cnt_hbm, dst_v, cnt_buf, cnt_sh, credbuf, credout):
  c = lax.axis_index("c")
  s = lax.axis_index("s")
  wid = c * NS + s
  pltpu.sync_copy(dst_hbm.at[wid], dst_v)
  zero16 = jnp.zeros((L,), jnp.float32)

  def zstep(i, carry):
    cnt_buf[pl.ds(i * L, L)] = zero16
    return carry

  lax.fori_loop(0, NR // L, zstep, 0)
  one16 = jnp.ones((L,), jnp.float32)

  def cstep(i, carry):
    j = i // (CH // L)
    k = i % (CH // L)
    dvec = dst_v[j, pl.ds(k * L, L)]
    plsc.addupdate_scatter(cnt_buf, [dvec], one16)
    return carry

  lax.fori_loop(0, EPW // L, cstep, 0)
  pltpu.sync_copy(cnt_buf, cnt_sh.at[s])
  plsc.subcore_barrier()

  pltpu.sync_copy(cnt_sh.at[:, pl.ds(s * STRIPE, STRIPE)], credbuf)

  def rstep(k, carry):
    a = credbuf[0, pl.ds(k * L, L)]
    for r in range(1, NS):
      a = a + credbuf[r, pl.ds(k * L, L)]
    credout[pl.ds(k * L, L)] = a
    return carry

  lax.fori_loop(0, STRIPE // L, rstep, 0)
  pltpu.sync_copy(credout, cnt_hbm.at[c, pl.ds(s * STRIPE, STRIPE)])


_sc_counts = pl.kernel(
    _sc_counts_body,
    out_type=[jax.ShapeDtypeStruct((NC, NR), jnp.float32)],
    mesh=_MESH,
    scratch_types=[
        pltpu.VMEM((NCHC, CH), jnp.int32),
        pltpu.VMEM((NR,), jnp.float32),
        pltpu.VMEM_SHARED((NS, NR), jnp.float32),
        pltpu.VMEM((NS, STRIPE), jnp.float32),
        pltpu.VMEM((STRIPE,), jnp.float32),
    ],
    compiler_params=pltpu.CompilerParams(needs_layout_passes=False))

BR = 1024
GRID = NR // BR


def _dense1_body(part, cnt, x, wl, wr, b, out):
  seg = part[0] + part[1]
  cv = cnt[...]
  ctot = jnp.maximum(cv[0] + cv[1], 1.0)
  mean = seg / ctot[:, None]
  h = jnp.dot(mean, wl[...], preferred_element_type=jnp.float32)
  h = h + jnp.dot(x[...], wr[...], preferred_element_type=jnp.float32)
  h = h + b[...]
  out[...] = jnp.maximum(h, 0.0)


def _dense2_body(part, cnt, h1, res, wl, wr, b, out):
  seg = part[0] + part[1]
  cv = cnt[...]
  ctot = jnp.maximum(cv[0] + cv[1], 1.0)
  mean = seg / ctot[:, None]
  h = jnp.dot(mean, wl[...], preferred_element_type=jnp.float32)
  h = h + jnp.dot(h1[...], wr[...], preferred_element_type=jnp.float32)
  h = h + b[...]
  out[...] = jnp.maximum(h, 0.0) + res[...]


_part_spec = pl.BlockSpec((NC, BR, D), lambda i: (0, i, 0))
_cnt_spec = pl.BlockSpec((NC, BR), lambda i: (0, i))
_row_spec = pl.BlockSpec((BR, D), lambda i: (i, 0))
_w_spec = pl.BlockSpec((D, D), lambda i: (0, 0))
_b_spec = pl.BlockSpec((1, D), lambda i: (0, 0))

_dense1 = pl.pallas_call(
    _dense1_body,
    grid=(GRID,),
    in_specs=[_part_spec, _cnt_spec, _row_spec, _w_spec, _w_spec, _b_spec],
    out_specs=_row_spec,
    out_shape=jax.ShapeDtypeStruct((N, D), jnp.float32),
)

_dense2 = pl.pallas_call(
    _dense2_body,
    grid=(GRID,),
    in_specs=[_part_spec, _cnt_spec, _row_spec, _row_spec, _w_spec, _w_spec,
              _b_spec],
    out_specs=_row_spec,
    out_shape=jax.ShapeDtypeStruct((N, D), jnp.float32),
)


def kernel(x, edge_index, W1l, b1, W1r, W2l, b2, W2r, g1, be1, g2, be2):
  s1 = g1 / jnp.sqrt(1.0 + EPS)
  s2 = g2 / jnp.sqrt(1.0 + EPS)
  w1l = W1l * s1[None, :]
  w1r = W1r * s1[None, :]
  bb1 = (b1 * s1 + be1)[None, :]
  w2l = W2l * s2[None, :]
  w2r = W2r * s2[None, :]
  bb2 = (b2 * s2 + be2)[None, :]

  src = edge_index[0]
  dst = edge_index[1]
  pad = NW * EPW - E
  src_p = jnp.concatenate([src, jnp.zeros((pad,), jnp.int32)])
  dst_p = jnp.concatenate([dst, N + (jnp.arange(pad, dtype=jnp.int32) % L)])
  src_f = src_p.reshape(TOTCH, CH)
  dst_f = dst_p.reshape(TOTCH, CH)
  dst_c = dst_p.reshape(NW, NCHC, CH)
  zrows = jnp.zeros((NR, D), jnp.float32)

  (cnt,) = _sc_counts(dst_c)
  (part1,) = _sc_scatter(x, src_f, dst_f, zrows)
  h1 = _dense1(part1, cnt, x, w1l, w1r, bb1)
  (part2,) = _sc_scatter(h1, src_f, dst_f, zrows)
  out = _dense2(part2, cnt, h1, x, w2l, w2r, bb2)
  return out
